# baseline (speedup 1.0000x reference)
import functools

import jax
import jax.numpy as jnp
from jax import lax
from jax.experimental import pallas as pl
from jax.experimental.pallas import tpu as pltpu
from jax.experimental.pallas import tpu_sc as plsc

NU = 10000
NI = 10000
E = 160000
D = 128

NC = 2
NS = 16
NW = NC * NS
L = 16

CHUNK = 128
NCH2 = 1280
EPAD = NCH2 * CHUNK
CPW = NCH2 // NW
CPW0 = 56
CPW1 = 24
NBUF = 2
LOOK = 1
NP = 10240
NPA = 10112
RB = 1280
RPT = NPA // NS
ZR = 16


def _mesh():
    return plsc.VectorSubcoreMesh(core_axis_name="c", subcore_axis_name="s")


def _deg_body(ts, td, bs, bd, rs, rd, out, deg_v, idx_v):
    c = lax.axis_index("c")
    s = lax.axis_index("s")
    wid = s * NC + c
    start = wid * CPW
    zero = jnp.zeros((L,), jnp.float32)

    def zb(j, _):
        deg_v[pl.ds(j * L, L)] = zero
        return 0
    lax.fori_loop(0, 6 * NP // L, zb, 0, unroll=8)
    ones = jnp.full((L,), 1.0, jnp.float32)
    for rel, src in enumerate((ts, td, bs, bd, rs, rd)):
        off = jnp.full((L,), rel * NP, jnp.int32)
        pltpu.sync_copy(src.at[pl.ds(start, CPW)], idx_v)

        def cb(i, _, off=off):
            for g in range(CHUNK // L):
                iv = idx_v[i, pl.ds(g * L, L)]
                plsc.addupdate_scatter(deg_v, [iv + off], ones)
            return 0
        lax.fori_loop(0, CPW, cb, 0)
    pltpu.sync_copy(deg_v, out.at[wid])


def _deg_call(ts, td, bs, bd, rs, rd):
    f = functools.partial(
        pl.kernel,
        out_type=jax.ShapeDtypeStruct((NW, 6 * NP), jnp.float32),
        mesh=_mesh(),
        compiler_params=pltpu.CompilerParams(needs_layout_passes=False),
        scratch_types=[
            pltpu.VMEM((6 * NP,), jnp.float32),
            pltpu.VMEM((CPW, CHUNK), jnp.int32),
        ],
    )(_deg_body)
    return f(ts, td, bs, bd, rs, rd)


def _spmm_body(yt, yb, yr, ts, td, bs, bd, rs, rd, cf, out,
               acc, sidx, didx, coefb, r0, r1, g0, g1, zsem):
    rows = (r0, r1)
    gsem = (g0, g1)
    c = lax.axis_index("c")
    s = lax.axis_index("s")
    wid = s * NC + c
    zvec = jnp.zeros((L,), jnp.float32)

    def scale_rows(buf, j):
        def sg(g, _):
            cvec = coefb[j, pl.ds(g * L, L)]
            for lane in range(L):
                e = g * L + lane
                bv = lax.broadcast(cvec[lane], (L,))
                for k in range(D // L):
                    sl = pl.ds(k * L, L)
                    buf[e, sl] = buf[e, sl] * bv
            return 0
        lax.fori_loop(0, CHUNK // L, sg, 0)

    def edge_loop(base, y, srch, dsth, cfh):
        pltpu.sync_copy(srch.at[pl.ds(base, CPW)], sidx)
        pltpu.sync_copy(dsth.at[pl.ds(base, CPW)], didx)
        if cfh is not None:
            pltpu.sync_copy(cfh.at[pl.ds(base, CPW)], coefb)

        def gather_desc(i, b):
            return pltpu.make_async_copy(y.at[sidx.at[i]], rows[b], gsem[b])

        def cb(j, _):
            pltpu.async_copy(y.at[sidx.at[j]], rows[0], gsem[0]).wait()
            if cfh is not None:
                scale_rows(rows[0], j)
            pltpu.sync_copy(rows[0], acc.at[didx.at[j]], add=True)
            return 0
        lax.fori_loop(0, CPW, cb, 0)

    convs = ((yt, ts, td, cf), (yb, bs, bd, None), (yr, rs, rd, None))
    for ci, (y, srch, dsth, cfh) in enumerate(convs):
        def zb(j, _):
            for k in range(D // L):
                rows[0][j, pl.ds(k * L, L)] = zvec
            return 0
        lax.fori_loop(0, CHUNK, zb, 0, unroll=8)
        nfull = RPT // CHUNK
        for j in range(nfull):
            pltpu.sync_copy(rows[0], acc.at[pl.ds(s * RPT + j * CHUNK, CHUNK)])
        rem = RPT - nfull * CHUNK
        pltpu.sync_copy(rows[0].at[pl.ds(0, rem)],
                        acc.at[pl.ds(s * RPT + nfull * CHUNK, rem)])
        plsc.subcore_barrier()

        edge_loop(wid * CPW, y, srch, dsth, cfh)

        plsc.subcore_barrier()
        pltpu.sync_copy(acc.at[pl.ds(s * RPT, RPT)],
                        out.at[ci, c, pl.ds(s * RPT, RPT)])
        plsc.subcore_barrier()


def _spmm_call(yt, yb, yr, ts, td, bs, bd, rs, rd, cf):
    f = functools.partial(
        pl.kernel,
        out_type=jax.ShapeDtypeStruct((3, NC, NPA, D), jnp.float32),
        mesh=_mesh(),
        compiler_params=pltpu.CompilerParams(needs_layout_passes=False),
        scratch_types=(
            [
                pltpu.VMEM_SHARED((NPA, D), jnp.float32),
                pltpu.VMEM((CPW, CHUNK), jnp.int32),
                pltpu.VMEM((CPW, CHUNK), jnp.int32),
                pltpu.VMEM((CPW, CHUNK), jnp.float32),
            ]
            + [pltpu.VMEM((CHUNK, D), jnp.float32)] * NBUF
            + [pltpu.SemaphoreType.DMA] * (NBUF + 1)
        ),
    )(_spmm_body)
    return f(yt, yb, yr, ts, td, bs, bd, rs, rd, cf)


def _norms_body(degp_ref, out_ref):
    dsum = jnp.sum(degp_ref[...], axis=0)
    out_ref[...] = jnp.where(dsum > 0, lax.rsqrt(dsum), 0.0)


def _norms_call(degp):
    return pl.pallas_call(
        _norms_body,
        grid=(NP // RB,),
        in_specs=[pl.BlockSpec((NW, 6, RB), lambda i: (0, 0, i))],
        out_specs=pl.BlockSpec((6, RB), lambda i: (0, i)),
        out_shape=jax.ShapeDtypeStruct((6, NP), jnp.float32),
    )(degp)


def _a_body(nrm_ref, fu_ref, fi_ref, wt_ref, wb_ref, wr_ref,
            yt_ref, yb_ref, yr_ref):
    nst = jnp.reshape(nrm_ref[0, :], (RB, 1))
    nsb = jnp.reshape(nrm_ref[2, :], (RB, 1))
    nsr = jnp.reshape(nrm_ref[4, :], (RB, 1))
    fu = fu_ref[...]
    fi = fi_ref[...]
    dot = functools.partial(jnp.dot, preferred_element_type=jnp.float32,
                            precision=lax.Precision.HIGHEST)
    yt_ref[...] = dot(fu * nst, wt_ref[...])
    yb_ref[...] = dot(fi * nsb, wb_ref[...])
    yr_ref[...] = dot(fu * nsr, wr_ref[...])


def _a_call(nrm, fu, fi, wt, wb, wr):
    g = NP // RB
    return pl.pallas_call(
        _a_body,
        grid=(g,),
        in_specs=[
            pl.BlockSpec((6, RB), lambda i: (0, i)),
            pl.BlockSpec((RB, D), lambda i: (i, 0)),
            pl.BlockSpec((RB, D), lambda i: (i, 0)),
            pl.BlockSpec((D, D), lambda i: (0, 0)),
            pl.BlockSpec((D, D), lambda i: (0, 0)),
            pl.BlockSpec((D, D), lambda i: (0, 0)),
        ],
        out_specs=[pl.BlockSpec((RB, D), lambda i: (i, 0))] * 3,
        out_shape=[jax.ShapeDtypeStruct((NPA, D), jnp.float32)] * 3,
    )(nrm, fu, fi, wt, wb, wr)


def _post(agg_ref, nrm_ref, b_ref):
    ndt = jnp.reshape(nrm_ref[1, :], (RB, 1))
    ndb = jnp.reshape(nrm_ref[3, :], (RB, 1))
    ndr = jnp.reshape(nrm_ref[5, :], (RB, 1))
    at = agg_ref[0, 0] + agg_ref[0, 1]
    ab = agg_ref[1, 0] + agg_ref[1, 1]
    ar = agg_ref[2, 0] + agg_ref[2, 1]
    ht = jnp.maximum(at * ndt + b_ref[0, :], 0.0)
    hb = jnp.maximum(ab * ndb + b_ref[1, :], 0.0)
    hr = jnp.maximum(ar * ndr + b_ref[2, :], 0.0)
    fu = (ht + hb) * 0.5
    return fu, hr


def _ba_body(agg_ref, nrm_ref, b_ref, wt_ref, wb_ref, wr_ref,
             fu_ref, fi_ref, yt_ref, yb_ref, yr_ref):
    fu, fi = _post(agg_ref, nrm_ref, b_ref)
    fu_ref[...] = fu
    fi_ref[...] = fi
    nst = jnp.reshape(nrm_ref[0, :], (RB, 1))
    nsb = jnp.reshape(nrm_ref[2, :], (RB, 1))
    nsr = jnp.reshape(nrm_ref[4, :], (RB, 1))
    dot = functools.partial(jnp.dot, preferred_element_type=jnp.float32,
                            precision=lax.Precision.HIGHEST)
    yt_ref[...] = dot(fu * nst, wt_ref[...])
    yb_ref[...] = dot(fi * nsb, wb_ref[...])
    yr_ref[...] = dot(fu * nsr, wr_ref[...])


def _ba_call(agg, nrm, b, wt, wb, wr):
    g = NP // RB
    return pl.pallas_call(
        _ba_body,
        grid=(g,),
        in_specs=[
            pl.BlockSpec((3, NC, RB, D), lambda i: (0, 0, i, 0)),
            pl.BlockSpec((6, RB), lambda i: (0, i)),
            pl.BlockSpec((3, D), lambda i: (0, 0)),
            pl.BlockSpec((D, D), lambda i: (0, 0)),
            pl.BlockSpec((D, D), lambda i: (0, 0)),
            pl.BlockSpec((D, D), lambda i: (0, 0)),
        ],
        out_specs=[pl.BlockSpec((RB, D), lambda i: (i, 0))] * 5,
        out_shape=([jax.ShapeDtypeStruct((NU, D), jnp.float32)] * 2
                   + [jax.ShapeDtypeStruct((NPA, D), jnp.float32)] * 3),
    )(agg, nrm, b, wt, wb, wr)


def _b_body(agg_ref, nrm_ref, b_ref, fu_ref, fi_ref):
    fu, fi = _post(agg_ref, nrm_ref, b_ref)
    fu_ref[...] = fu
    fi_ref[...] = fi


def _b_call(agg, nrm, b):
    g = NP // RB
    return pl.pallas_call(
        _b_body,
        grid=(g,),
        in_specs=[
            pl.BlockSpec((3, NC, RB, D), lambda i: (0, 0, i, 0)),
            pl.BlockSpec((6, RB), lambda i: (0, i)),
            pl.BlockSpec((3, D), lambda i: (0, 0)),
        ],
        out_specs=[pl.BlockSpec((RB, D), lambda i: (i, 0))] * 2,
        out_shape=[jax.ShapeDtypeStruct((NU, D), jnp.float32)] * 2,
    )(agg, nrm, b)


def kernel(norm_edge_weight, user_embeddings, item_embeddings,
           Wt0, bt0, Wr0, br0, Wb0, bb0, Wt1, bt1, Wr1, br1, Wb1, bb1,
           trust_src, trust_dst, rate_src, rate_dst, rb_src, rb_dst):
    pad = jnp.full((EPAD - E,), NU, jnp.int32)

    def idx(a):
        return jnp.concatenate([a.astype(jnp.int32), pad]).reshape(NCH2, CHUNK)
    ts, td = idx(trust_src), idx(trust_dst)
    bs, bd = idx(rb_src), idx(rb_dst)
    rs, rd = idx(rate_src), idx(rate_dst)
    cf = jnp.concatenate(
        [norm_edge_weight, jnp.zeros((EPAD - E,), jnp.float32)]
    ).reshape(NCH2, CHUNK)

    degp = _deg_call(ts, td, bs, bd, rs, rd).reshape(NW, 6, NP)
    nrm = _norms_call(degp)

    yt0, yb0, yr0 = _a_call(nrm, user_embeddings, item_embeddings,
                            Wt0, Wb0, Wr0)
    agg0 = _spmm_call(yt0, yb0, yr0, ts, td, bs, bd, rs, rd, cf)
    b0 = jnp.stack([bt0, bb0, br0])
    fu1, fi1, yt1, yb1, yr1 = _ba_call(agg0, nrm, b0, Wt1, Wb1, Wr1)

    agg1 = _spmm_call(yt1, yb1, yr1, ts, td, bs, bd, rs, rd, cf)
    b1 = jnp.stack([bt1, bb1, br1])
    fu2, fi2 = _b_call(agg1, nrm, b1)

    out_u = jnp.concatenate([user_embeddings, fu1, fu2], axis=1)
    out_i = jnp.concatenate([item_embeddings, fi1, fi2], axis=1)
    return (out_u, out_i)


# speedup vs baseline: 1.0295x; 1.0295x over previous
import functools

import jax
import jax.numpy as jnp
from jax import lax
from jax.experimental import pallas as pl
from jax.experimental.pallas import tpu as pltpu
from jax.experimental.pallas import tpu_sc as plsc

NU = 10000
NI = 10000
E = 160000
D = 128

NC = 2
NS = 16
NW = NC * NS
L = 16

CHUNK = 128
NCHUNK = E // CHUNK
NCH2 = 1280
EPAD = NCH2 * CHUNK
CPW = NCH2 // NW
NP = 10240
RB = 1280
RPT = NP // NS
ZR = 128


def _mesh():
    return plsc.VectorSubcoreMesh(core_axis_name="c", subcore_axis_name="s")


def _deg_body(ts, td, bs, bd, rs, rd, out, deg_v, idx_v):
    c = lax.axis_index("c")
    s = lax.axis_index("s")
    wid = s * NC + c
    start = wid * CPW
    zero = jnp.zeros((L,), jnp.float32)

    def zb(j, _):
        deg_v[pl.ds(j * L, L)] = zero
        return 0
    lax.fori_loop(0, 6 * NP // L, zb, 0, unroll=8)
    ones = jnp.full((L,), 1.0, jnp.float32)
    for rel, src in enumerate((ts, td, bs, bd, rs, rd)):
        off = jnp.full((L,), rel * NP, jnp.int32)
        pltpu.sync_copy(src.at[pl.ds(start, CPW)], idx_v)

        def cb(i, _, off=off):
            for g in range(CHUNK // L):
                iv = idx_v[i, pl.ds(g * L, L)]
                plsc.addupdate_scatter(deg_v, [iv + off], ones)
            return 0
        lax.fori_loop(0, CPW, cb, 0)
    pltpu.sync_copy(deg_v, out.at[wid])


def _deg_call(ts, td, bs, bd, rs, rd):
    f = functools.partial(
        pl.kernel,
        out_type=jax.ShapeDtypeStruct((NW, 6 * NP), jnp.float32),
        mesh=_mesh(),
        compiler_params=pltpu.CompilerParams(needs_layout_passes=False),
        scratch_types=[
            pltpu.VMEM((6 * NP,), jnp.float32),
            pltpu.VMEM((CPW, CHUNK), jnp.int32),
        ],
    )(_deg_body)
    return f(ts, td, bs, bd, rs, rd)


def _spmm_body(yt, yb, yr, ts, td, bs, bd, rs, rd, cf, out,
               acc, rows_v, sidx_v, didx_v, coef_v, zero_v, sem):
    c = lax.axis_index("c")
    s = lax.axis_index("s")
    wid = s * NC + c
    zvec = jnp.zeros((L,), jnp.float32)

    def zb(j, _):
        zero_v[j // (D // L), pl.ds((j % (D // L)) * L, L)] = zvec
        return 0
    lax.fori_loop(0, ZR * (D // L), zb, 0)

    nmy = CPW
    convs = ((yt, ts, td, cf), (yb, bs, bd, None), (yr, rs, rd, None))
    for ci, (y, srch, dsth, cfh) in enumerate(convs):
        for j in range(RPT // ZR):
            pltpu.sync_copy(zero_v, acc.at[pl.ds(s * RPT + j * ZR, ZR)])
        plsc.subcore_barrier()

        def eb(i, _, y=y, srch=srch, dsth=dsth, cfh=cfh):
            chunk = wid + i * NW
            pltpu.sync_copy(srch.at[chunk], sidx_v)
            pltpu.sync_copy(dsth.at[chunk], didx_v.at[0])
            pltpu.async_copy(y.at[sidx_v], rows_v, sem).wait()
            if cfh is not None:
                pltpu.sync_copy(cfh.at[chunk], coef_v)

                def sb(g, _):
                    cvec = coef_v[pl.ds(g * L, L)]
                    for j in range(L):
                        e = g * L + j
                        bv = lax.broadcast(cvec[j], (L,))
                        for k in range(D // L):
                            sl = pl.ds(k * L, L)
                            rows_v[e, sl] = rows_v[e, sl] * bv
                    return 0
                lax.fori_loop(0, CHUNK // L, sb, 0)
            pltpu.sync_copy(rows_v, acc.at[didx_v.at[0]], add=True)
            return 0
        lax.fori_loop(0, nmy, eb, 0)
        plsc.subcore_barrier()
        pltpu.sync_copy(acc.at[pl.ds(s * RPT, RPT)],
                        out.at[ci, c, pl.ds(s * RPT, RPT)])
        plsc.subcore_barrier()


def _spmm_call(yt, yb, yr, ts, td, bs, bd, rs, rd, cf):
    f = functools.partial(
        pl.kernel,
        out_type=jax.ShapeDtypeStruct((3, NC, NP, D), jnp.float32),
        mesh=_mesh(),
        compiler_params=pltpu.CompilerParams(needs_layout_passes=False),
        scratch_types=[
            pltpu.VMEM_SHARED((NP, D), jnp.float32),
            pltpu.VMEM((CHUNK, D), jnp.float32),
            pltpu.VMEM((CHUNK,), jnp.int32),
            pltpu.VMEM((1, CHUNK), jnp.int32),
            pltpu.VMEM((CHUNK,), jnp.float32),
            pltpu.VMEM((ZR, D), jnp.float32),
            pltpu.SemaphoreType.DMA,
        ],
    )(_spmm_body)
    return f(yt, yb, yr, ts, td, bs, bd, rs, rd, cf)


def _norms_body(degp_ref, out_ref):
    dsum = jnp.sum(degp_ref[...], axis=0)
    out_ref[...] = jnp.where(dsum > 0, lax.rsqrt(dsum), 0.0)


def _norms_call(degp):
    return pl.pallas_call(
        _norms_body,
        grid=(NP // RB,),
        in_specs=[pl.BlockSpec((NW, 6, RB), lambda i: (0, 0, i))],
        out_specs=pl.BlockSpec((6, RB), lambda i: (0, i)),
        out_shape=jax.ShapeDtypeStruct((6, NP), jnp.float32),
    )(degp)


def _a_body(nrm_ref, fu_ref, fi_ref, wt_ref, wb_ref, wr_ref,
            yt_ref, yb_ref, yr_ref):
    nst = jnp.reshape(nrm_ref[0, :], (RB, 1))
    nsb = jnp.reshape(nrm_ref[2, :], (RB, 1))
    nsr = jnp.reshape(nrm_ref[4, :], (RB, 1))
    fu = fu_ref[...]
    fi = fi_ref[...]
    dot = functools.partial(jnp.dot, preferred_element_type=jnp.float32,
                            precision=lax.Precision.HIGHEST)
    yt_ref[...] = dot(fu * nst, wt_ref[...])
    yb_ref[...] = dot(fi * nsb, wb_ref[...])
    yr_ref[...] = dot(fu * nsr, wr_ref[...])


def _a_call(nrm, fu, fi, wt, wb, wr):
    g = NP // RB
    return pl.pallas_call(
        _a_body,
        grid=(g,),
        in_specs=[
            pl.BlockSpec((6, RB), lambda i: (0, i)),
            pl.BlockSpec((RB, D), lambda i: (i, 0)),
            pl.BlockSpec((RB, D), lambda i: (i, 0)),
            pl.BlockSpec((D, D), lambda i: (0, 0)),
            pl.BlockSpec((D, D), lambda i: (0, 0)),
            pl.BlockSpec((D, D), lambda i: (0, 0)),
        ],
        out_specs=[pl.BlockSpec((RB, D), lambda i: (i, 0))] * 3,
        out_shape=[jax.ShapeDtypeStruct((NP, D), jnp.float32)] * 3,
    )(nrm, fu, fi, wt, wb, wr)


def _post(agg_ref, nrm_ref, b_ref):
    ndt = jnp.reshape(nrm_ref[1, :], (RB, 1))
    ndb = jnp.reshape(nrm_ref[3, :], (RB, 1))
    ndr = jnp.reshape(nrm_ref[5, :], (RB, 1))
    at = agg_ref[0, 0] + agg_ref[0, 1]
    ab = agg_ref[1, 0] + agg_ref[1, 1]
    ar = agg_ref[2, 0] + agg_ref[2, 1]
    ht = jnp.maximum(at * ndt + b_ref[0, :], 0.0)
    hb = jnp.maximum(ab * ndb + b_ref[1, :], 0.0)
    hr = jnp.maximum(ar * ndr + b_ref[2, :], 0.0)
    fu = (ht + hb) * 0.5
    return fu, hr


def _ba_body(agg_ref, nrm_ref, b_ref, wt_ref, wb_ref, wr_ref,
             fu_ref, fi_ref, yt_ref, yb_ref, yr_ref):
    fu, fi = _post(agg_ref, nrm_ref, b_ref)
    fu_ref[...] = fu
    fi_ref[...] = fi
    nst = jnp.reshape(nrm_ref[0, :], (RB, 1))
    nsb = jnp.reshape(nrm_ref[2, :], (RB, 1))
    nsr = jnp.reshape(nrm_ref[4, :], (RB, 1))
    dot = functools.partial(jnp.dot, preferred_element_type=jnp.float32,
                            precision=lax.Precision.HIGHEST)
    yt_ref[...] = dot(fu * nst, wt_ref[...])
    yb_ref[...] = dot(fi * nsb, wb_ref[...])
    yr_ref[...] = dot(fu * nsr, wr_ref[...])


def _ba_call(agg, nrm, b, wt, wb, wr):
    g = NP // RB
    return pl.pallas_call(
        _ba_body,
        grid=(g,),
        in_specs=[
            pl.BlockSpec((3, NC, RB, D), lambda i: (0, 0, i, 0)),
            pl.BlockSpec((6, RB), lambda i: (0, i)),
            pl.BlockSpec((3, D), lambda i: (0, 0)),
            pl.BlockSpec((D, D), lambda i: (0, 0)),
            pl.BlockSpec((D, D), lambda i: (0, 0)),
            pl.BlockSpec((D, D), lambda i: (0, 0)),
        ],
        out_specs=[pl.BlockSpec((RB, D), lambda i: (i, 0))] * 5,
        out_shape=([jax.ShapeDtypeStruct((NU, D), jnp.float32)] * 2
                   + [jax.ShapeDtypeStruct((NP, D), jnp.float32)] * 3),
    )(agg, nrm, b, wt, wb, wr)


def _b_body(agg_ref, nrm_ref, b_ref, fu_ref, fi_ref):
    fu, fi = _post(agg_ref, nrm_ref, b_ref)
    fu_ref[...] = fu
    fi_ref[...] = fi


def _b_call(agg, nrm, b):
    g = NP // RB
    return pl.pallas_call(
        _b_body,
        grid=(g,),
        in_specs=[
            pl.BlockSpec((3, NC, RB, D), lambda i: (0, 0, i, 0)),
            pl.BlockSpec((6, RB), lambda i: (0, i)),
            pl.BlockSpec((3, D), lambda i: (0, 0)),
        ],
        out_specs=[pl.BlockSpec((RB, D), lambda i: (i, 0))] * 2,
        out_shape=[jax.ShapeDtypeStruct((NU, D), jnp.float32)] * 2,
    )(agg, nrm, b)


def kernel(norm_edge_weight, user_embeddings, item_embeddings,
           Wt0, bt0, Wr0, br0, Wb0, bb0, Wt1, bt1, Wr1, br1, Wb1, bb1,
           trust_src, trust_dst, rate_src, rate_dst, rb_src, rb_dst):
    pad = jnp.full((EPAD - E,), NU, jnp.int32)

    def idx(a):
        return jnp.concatenate([a.astype(jnp.int32), pad]).reshape(NCH2, CHUNK)
    ts, td = idx(trust_src), idx(trust_dst)
    bs, bd = idx(rb_src), idx(rb_dst)
    rs, rd = idx(rate_src), idx(rate_dst)
    cf = jnp.concatenate(
        [norm_edge_weight, jnp.zeros((EPAD - E,), jnp.float32)]
    ).reshape(NCH2, CHUNK)

    degp = _deg_call(ts, td, bs, bd, rs, rd).reshape(NW, 6, NP)
    nrm = _norms_call(degp)

    yt0, yb0, yr0 = _a_call(nrm, user_embeddings, item_embeddings,
                            Wt0, Wb0, Wr0)
    agg0 = _spmm_call(yt0, yb0, yr0, ts, td, bs, bd, rs, rd, cf)
    b0 = jnp.stack([bt0, bb0, br0])
    fu1, fi1, yt1, yb1, yr1 = _ba_call(agg0, nrm, b0, Wt1, Wb1, Wr1)

    agg1 = _spmm_call(yt1, yb1, yr1, ts, td, bs, bd, rs, rd, cf)
    b1 = jnp.stack([bt1, bb1, br1])
    fu2, fi2 = _b_call(agg1, nrm, b1)

    out_u = jnp.concatenate([user_embeddings, fu1, fu2], axis=1)
    out_i = jnp.concatenate([item_embeddings, fi1, fi2], axis=1)
    return (out_u, out_i)


# speedup vs baseline: 1.8979x; 1.8435x over previous
import functools

import jax
import jax.numpy as jnp
from jax import lax
from jax.experimental import pallas as pl
from jax.experimental.pallas import tpu as pltpu
from jax.experimental.pallas import tpu_sc as plsc

NU = 10000
NI = 10000
E = 160000
D = 128

NC = 2
NS = 16
NW = NC * NS
L = 16

CHUNK = 128
NCHUNK = E // CHUNK
NCH2 = 1280
EPAD = NCH2 * CHUNK
CPW = NCH2 // NW
NP = 10240
RB = 1280
RPT = NP // NS
ZR = 128


def _mesh():
    return plsc.VectorSubcoreMesh(core_axis_name="c", subcore_axis_name="s")


def _deg_body(ts, td, bs, bd, rs, rd, out, deg_v, idx_v):
    c = lax.axis_index("c")
    s = lax.axis_index("s")
    wid = s * NC + c
    start = wid * CPW
    zero = jnp.zeros((L,), jnp.float32)

    def zb(j, _):
        deg_v[pl.ds(j * L, L)] = zero
        return 0
    lax.fori_loop(0, 6 * NP // L, zb, 0, unroll=8)
    ones = jnp.full((L,), 1.0, jnp.float32)
    for rel, src in enumerate((ts, td, bs, bd, rs, rd)):
        off = jnp.full((L,), rel * NP, jnp.int32)
        pltpu.sync_copy(src.at[pl.ds(start, CPW)], idx_v)

        def cb(i, _, off=off):
            for g in range(CHUNK // L):
                iv = idx_v[i, pl.ds(g * L, L)]
                plsc.addupdate_scatter(deg_v, [iv + off], ones)
            return 0
        lax.fori_loop(0, CPW, cb, 0)
    pltpu.sync_copy(deg_v, out.at[wid])


def _deg_call(ts, td, bs, bd, rs, rd):
    f = functools.partial(
        pl.kernel,
        out_type=jax.ShapeDtypeStruct((NW, 6 * NP), jnp.float32),
        mesh=_mesh(),
        compiler_params=pltpu.CompilerParams(needs_layout_passes=False),
        scratch_types=[
            pltpu.VMEM((6 * NP,), jnp.float32),
            pltpu.VMEM((CPW, CHUNK), jnp.int32),
        ],
    )(_deg_body)
    return f(ts, td, bs, bd, rs, rd)


def _spmm_body(yt, yb, yr, ts, td, bs, bd, rs, rd, cf, out,
               acc, rows_v, sidx_v, didx_v, coef_v, zero_v, sem):
    c = lax.axis_index("c")
    s = lax.axis_index("s")
    wid = s * NC + c
    zvec = jnp.zeros((L,), jnp.float32)

    def zb(j, _):
        zero_v[j // (D // L), pl.ds((j % (D // L)) * L, L)] = zvec
        return 0
    lax.fori_loop(0, ZR * (D // L), zb, 0)

    nmy = CPW
    convs = ((yt, ts, td, cf), (yb, bs, bd, None), (yr, rs, rd, None))
    for ci, (y, srch, dsth, cfh) in enumerate(convs):
        for j in range(RPT // ZR):
            pltpu.sync_copy(zero_v, acc.at[pl.ds(s * RPT + j * ZR, ZR)])
        plsc.subcore_barrier()

        def eb(i, _, y=y, srch=srch, dsth=dsth, cfh=cfh):
            chunk = wid + i * NW
            pltpu.sync_copy(srch.at[chunk], sidx_v)
            pltpu.sync_copy(dsth.at[chunk], didx_v.at[0])
            pltpu.async_copy(y.at[sidx_v], rows_v, sem).wait()
            if cfh is not None:
                pltpu.sync_copy(cfh.at[chunk], coef_v)

                def sb(g, _):
                    cvec = coef_v[pl.ds(g * L, L)]
                    for j in range(L):
                        e = g * L + j
                        bv = lax.broadcast(cvec[j], (L,))
                        for k in range(D // L):
                            sl = pl.ds(k * L, L)
                            rows_v[e, sl] = rows_v[e, sl] * bv
                    return 0
                lax.fori_loop(0, CHUNK // L, sb, 0)
            pltpu.sync_copy(rows_v, acc.at[didx_v.at[0]], add=True)
            return 0
        lax.fori_loop(0, nmy, eb, 0)
        plsc.subcore_barrier()
        pltpu.sync_copy(acc.at[pl.ds(s * RPT, RPT)],
                        out.at[ci, c, pl.ds(s * RPT, RPT)])
        plsc.subcore_barrier()


def _spmm_call(yt, yb, yr, ts, td, bs, bd, rs, rd, cf):
    f = functools.partial(
        pl.kernel,
        out_type=jax.ShapeDtypeStruct((3, NC, NP, D), jnp.float32),
        mesh=_mesh(),
        compiler_params=pltpu.CompilerParams(needs_layout_passes=False),
        scratch_types=[
            pltpu.VMEM_SHARED((NP, D), jnp.float32),
            pltpu.VMEM((CHUNK, D), jnp.float32),
            pltpu.VMEM((CHUNK,), jnp.int32),
            pltpu.VMEM((1, CHUNK), jnp.int32),
            pltpu.VMEM((CHUNK,), jnp.float32),
            pltpu.VMEM((ZR, D), jnp.float32),
            pltpu.SemaphoreType.DMA,
        ],
    )(_spmm_body)
    return f(yt, yb, yr, ts, td, bs, bd, rs, rd, cf)


def _norms_body(degp_ref, out_ref):
    dsum = jnp.sum(degp_ref[...], axis=0)
    out_ref[...] = jnp.where(dsum > 0, lax.rsqrt(dsum), 0.0)


def _norms_call(degp):
    return pl.pallas_call(
        _norms_body,
        grid=(NP // RB,),
        in_specs=[pl.BlockSpec((NW, 6, RB), lambda i: (0, 0, i))],
        out_specs=pl.BlockSpec((6, RB), lambda i: (0, i)),
        out_shape=jax.ShapeDtypeStruct((6, NP), jnp.float32),
    )(degp)


def _a_body(nrm_ref, fu_ref, fi_ref, wt_ref, wb_ref, wr_ref,
            yt_ref, yb_ref, yr_ref):
    nst = jnp.reshape(nrm_ref[0, :], (RB, 1))
    nsb = jnp.reshape(nrm_ref[2, :], (RB, 1))
    nsr = jnp.reshape(nrm_ref[4, :], (RB, 1))
    fu = fu_ref[...]
    fi = fi_ref[...]
    dot = functools.partial(jnp.dot, preferred_element_type=jnp.float32,
                            precision=lax.Precision.HIGHEST)
    yt_ref[...] = dot(fu * nst, wt_ref[...])
    yb_ref[...] = dot(fi * nsb, wb_ref[...])
    yr_ref[...] = dot(fu * nsr, wr_ref[...])


def _a_call(nrm, fu, fi, wt, wb, wr):
    g = NP // RB
    return pl.pallas_call(
        _a_body,
        grid=(g,),
        in_specs=[
            pl.BlockSpec((6, RB), lambda i: (0, i)),
            pl.BlockSpec((RB, D), lambda i: (i, 0)),
            pl.BlockSpec((RB, D), lambda i: (i, 0)),
            pl.BlockSpec((D, D), lambda i: (0, 0)),
            pl.BlockSpec((D, D), lambda i: (0, 0)),
            pl.BlockSpec((D, D), lambda i: (0, 0)),
        ],
        out_specs=[pl.BlockSpec((RB, D), lambda i: (i, 0))] * 3,
        out_shape=[jax.ShapeDtypeStruct((NP, D), jnp.float32)] * 3,
    )(nrm, fu, fi, wt, wb, wr)


def _post(agg_ref, nrm_ref, b_ref):
    ndt = jnp.reshape(nrm_ref[1, :], (RB, 1))
    ndb = jnp.reshape(nrm_ref[3, :], (RB, 1))
    ndr = jnp.reshape(nrm_ref[5, :], (RB, 1))
    at = agg_ref[0, 0] + agg_ref[0, 1]
    ab = agg_ref[1, 0] + agg_ref[1, 1]
    ar = agg_ref[2, 0] + agg_ref[2, 1]
    ht = jnp.maximum(at * ndt + b_ref[0, :], 0.0)
    hb = jnp.maximum(ab * ndb + b_ref[1, :], 0.0)
    hr = jnp.maximum(ar * ndr + b_ref[2, :], 0.0)
    fu = (ht + hb) * 0.5
    return fu, hr


def _ba_body(agg_ref, nrm_ref, b_ref, wt_ref, wb_ref, wr_ref,
             fu_ref, fi_ref, yt_ref, yb_ref, yr_ref):
    fu, fi = _post(agg_ref, nrm_ref, b_ref)
    fu_ref[...] = fu
    fi_ref[...] = fi
    nst = jnp.reshape(nrm_ref[0, :], (RB, 1))
    nsb = jnp.reshape(nrm_ref[2, :], (RB, 1))
    nsr = jnp.reshape(nrm_ref[4, :], (RB, 1))
    dot = functools.partial(jnp.dot, preferred_element_type=jnp.float32,
                            precision=lax.Precision.HIGHEST)
    yt_ref[...] = dot(fu * nst, wt_ref[...])
    yb_ref[...] = dot(fi * nsb, wb_ref[...])
    yr_ref[...] = dot(fu * nsr, wr_ref[...])


def _ba_call(agg, nrm, b, wt, wb, wr):
    g = NP // RB
    return pl.pallas_call(
        _ba_body,
        grid=(g,),
        in_specs=[
            pl.BlockSpec((3, NC, RB, D), lambda i: (0, 0, i, 0)),
            pl.BlockSpec((6, RB), lambda i: (0, i)),
            pl.BlockSpec((3, D), lambda i: (0, 0)),
            pl.BlockSpec((D, D), lambda i: (0, 0)),
            pl.BlockSpec((D, D), lambda i: (0, 0)),
            pl.BlockSpec((D, D), lambda i: (0, 0)),
        ],
        out_specs=[pl.BlockSpec((RB, D), lambda i: (i, 0))] * 5,
        out_shape=([jax.ShapeDtypeStruct((NU, D), jnp.float32)] * 2
                   + [jax.ShapeDtypeStruct((NP, D), jnp.float32)] * 3),
    )(agg, nrm, b, wt, wb, wr)


def _b_body(agg_ref, nrm_ref, b_ref, fu_ref, fi_ref):
    fu, fi = _post(agg_ref, nrm_ref, b_ref)
    fu_ref[...] = fu
    fi_ref[...] = fi


def _b_call(agg, nrm, b):
    g = NP // RB
    return pl.pallas_call(
        _b_body,
        grid=(g,),
        in_specs=[
            pl.BlockSpec((3, NC, RB, D), lambda i: (0, 0, i, 0)),
            pl.BlockSpec((6, RB), lambda i: (0, i)),
            pl.BlockSpec((3, D), lambda i: (0, 0)),
        ],
        out_specs=[pl.BlockSpec((RB, D), lambda i: (i, 0))] * 2,
        out_shape=[jax.ShapeDtypeStruct((NU, D), jnp.float32)] * 2,
    )(agg, nrm, b)


def kernel(norm_edge_weight, user_embeddings, item_embeddings,
           Wt0, bt0, Wr0, br0, Wb0, bb0, Wt1, bt1, Wr1, br1, Wb1, bb1,
           trust_src, trust_dst, rate_src, rate_dst, rb_src, rb_dst):
    pad = NU + (jnp.arange(EPAD - E, dtype=jnp.int32) % (NP - NU))

    def idx(a):
        return jnp.concatenate([a.astype(jnp.int32), pad]).reshape(NCH2, CHUNK)
    ts, td = idx(trust_src), idx(trust_dst)
    bs, bd = idx(rb_src), idx(rb_dst)
    rs, rd = idx(rate_src), idx(rate_dst)
    cf = jnp.concatenate(
        [norm_edge_weight, jnp.zeros((EPAD - E,), jnp.float32)]
    ).reshape(NCH2, CHUNK)

    degp = _deg_call(ts, td, bs, bd, rs, rd).reshape(NW, 6, NP)
    nrm = _norms_call(degp)

    yt0, yb0, yr0 = _a_call(nrm, user_embeddings, item_embeddings,
                            Wt0, Wb0, Wr0)
    agg0 = _spmm_call(yt0, yb0, yr0, ts, td, bs, bd, rs, rd, cf)
    b0 = jnp.stack([bt0, bb0, br0])
    fu1, fi1, yt1, yb1, yr1 = _ba_call(agg0, nrm, b0, Wt1, Wb1, Wr1)

    agg1 = _spmm_call(yt1, yb1, yr1, ts, td, bs, bd, rs, rd, cf)
    b1 = jnp.stack([bt1, bb1, br1])
    fu2, fi2 = _b_call(agg1, nrm, b1)

    out_u = jnp.concatenate([user_embeddings, fu1, fu2], axis=1)
    out_i = jnp.concatenate([item_embeddings, fi1, fi2], axis=1)
    return (out_u, out_i)


# speedup vs baseline: 3.4400x; 1.8125x over previous
import functools

import jax
import jax.numpy as jnp
from jax import lax
from jax.experimental import pallas as pl
from jax.experimental.pallas import tpu as pltpu
from jax.experimental.pallas import tpu_sc as plsc

NU = 10000
NI = 10000
E = 160000
D = 128

NC = 2
NS = 16
NW = NC * NS
L = 16

CHUNK = 128
NCHUNK = E // CHUNK
NCH2 = 1280
EPAD = NCH2 * CHUNK
CPW = NCH2 // NW
NBUF = 2
LOOK = 1
ZR = 16
NP = 10240
RB = 1280
RPT = NP // NS
ZR = 128


def _mesh():
    return plsc.VectorSubcoreMesh(core_axis_name="c", subcore_axis_name="s")


def _deg_body(ts, td, bs, bd, rs, rd, out, deg_v, idx_v):
    c = lax.axis_index("c")
    s = lax.axis_index("s")
    wid = s * NC + c
    start = wid * CPW
    zero = jnp.zeros((L,), jnp.float32)

    def zb(j, _):
        deg_v[pl.ds(j * L, L)] = zero
        return 0
    lax.fori_loop(0, 6 * NP // L, zb, 0, unroll=8)
    ones = jnp.full((L,), 1.0, jnp.float32)
    for rel, src in enumerate((ts, td, bs, bd, rs, rd)):
        off = jnp.full((L,), rel * NP, jnp.int32)
        pltpu.sync_copy(src.at[pl.ds(start, CPW)], idx_v)

        def cb(i, _, off=off):
            for g in range(CHUNK // L):
                iv = idx_v[i, pl.ds(g * L, L)]
                plsc.addupdate_scatter(deg_v, [iv + off], ones)
            return 0
        lax.fori_loop(0, CPW, cb, 0)
    pltpu.sync_copy(deg_v, out.at[wid])


def _deg_call(ts, td, bs, bd, rs, rd):
    f = functools.partial(
        pl.kernel,
        out_type=jax.ShapeDtypeStruct((NW, 6 * NP), jnp.float32),
        mesh=_mesh(),
        compiler_params=pltpu.CompilerParams(needs_layout_passes=False),
        scratch_types=[
            pltpu.VMEM((6 * NP,), jnp.float32),
            pltpu.VMEM((CPW, CHUNK), jnp.int32),
        ],
    )(_deg_body)
    return f(ts, td, bs, bd, rs, rd)


def _spmm_body(yt, yb, yr, ts, td, bs, bd, rs, rd, cf, out,
               acc, sidx, didx, coefb2, r0, r1, g0, g1, s0, s1):
    rows = (r0, r1)
    gsem = (g0, g1)
    ssem = (s0, s1)
    c = lax.axis_index("c")
    s = lax.axis_index("s")
    wid = s * NC + c
    start = wid * CPW
    zvec = jnp.zeros((L,), jnp.float32)

    def scale_rows(buf):
        def sg(g, _):
            cvec = coefb2[pl.ds(g * L, L)]
            for lane in range(L):
                e = g * L + lane
                bv = lax.broadcast(cvec[lane], (L,))
                for k in range(D // L):
                    sl = pl.ds(k * L, L)
                    buf[e, sl] = buf[e, sl] * bv
            return 0
        lax.fori_loop(0, CHUNK // L, sg, 0)

    convs = ((yt, ts, td, cf), (yb, bs, bd, None), (yr, rs, rd, None))
    for ci, (y, srch, dsth, cfh) in enumerate(convs):
        def zrow(j, _):
            for k in range(D // L):
                rows[0][j, pl.ds(k * L, L)] = zvec
            return 0
        lax.fori_loop(0, CHUNK, zrow, 0, unroll=8)
        for j in range(RPT // CHUNK):
            pltpu.sync_copy(rows[0], acc.at[pl.ds(s * RPT + j * CHUNK, CHUNK)])
        pltpu.sync_copy(srch.at[pl.ds(start, CPW)], sidx)
        pltpu.sync_copy(dsth.at[pl.ds(start, CPW)], didx)
        plsc.subcore_barrier()

        def gather_desc(i, b, y=y):
            return pltpu.make_async_copy(y.at[sidx.at[i]], rows[b], gsem[b])

        def scatter_desc(j, b):
            return pltpu.make_async_copy(rows[b], acc.at[didx.at[j]], ssem[b])

        def consume(j, b, cfh=cfh):
            if cfh is not None:
                pltpu.sync_copy(cfh.at[start + j], coefb2)
            gather_desc(j, b).wait()
            if cfh is not None:
                scale_rows(rows[b])
            pltpu.async_copy(rows[b], acc.at[didx.at[j]], ssem[b], add=True)

        for b in range(LOOK):
            pltpu.async_copy(y.at[sidx.at[b]], rows[b], gsem[b])

        def group(g, _):
            for b in range(NBUF):
                i = g * NBUF + b + LOOK
                bi = (b + LOOK) % NBUF
                j = g * NBUF + b

                @pl.when(i < CPW)
                def _(i=i, bi=bi):
                    @pl.when(i >= NBUF)
                    def _(i=i, bi=bi):
                        scatter_desc(i - NBUF, bi).wait()
                    gather_desc(i, bi).start()
                consume(j, b)
            return 0
        lax.fori_loop(0, CPW // NBUF, group, 0)

        for j in range(CPW - NBUF, CPW):
            scatter_desc(j, j % NBUF).wait()
        plsc.subcore_barrier()
        pltpu.sync_copy(acc.at[pl.ds(s * RPT, RPT)],
                        out.at[ci, c, pl.ds(s * RPT, RPT)])
        plsc.subcore_barrier()


def _spmm_call(yt, yb, yr, ts, td, bs, bd, rs, rd, cf):
    f = functools.partial(
        pl.kernel,
        out_type=jax.ShapeDtypeStruct((3, NC, NP, D), jnp.float32),
        mesh=_mesh(),
        compiler_params=pltpu.CompilerParams(needs_layout_passes=False),
        scratch_types=(
            [
                pltpu.VMEM_SHARED((NP, D), jnp.float32),
                pltpu.VMEM((CPW, CHUNK), jnp.int32),
                pltpu.VMEM((CPW, CHUNK), jnp.int32),
                pltpu.VMEM((CHUNK,), jnp.float32),
            ]
            + [pltpu.VMEM((CHUNK, D), jnp.float32)] * NBUF
            + [pltpu.SemaphoreType.DMA] * (2 * NBUF)
        ),
    )(_spmm_body)
    return f(yt, yb, yr, ts, td, bs, bd, rs, rd, cf)


def _norms_body(degp_ref, out_ref):
    dsum = jnp.sum(degp_ref[...], axis=0)
    out_ref[...] = jnp.where(dsum > 0, lax.rsqrt(dsum), 0.0)


def _norms_call(degp):
    return pl.pallas_call(
        _norms_body,
        grid=(NP // RB,),
        in_specs=[pl.BlockSpec((NW, 6, RB), lambda i: (0, 0, i))],
        out_specs=pl.BlockSpec((6, RB), lambda i: (0, i)),
        out_shape=jax.ShapeDtypeStruct((6, NP), jnp.float32),
    )(degp)


def _a_body(nrm_ref, fu_ref, fi_ref, wt_ref, wb_ref, wr_ref,
            yt_ref, yb_ref, yr_ref):
    nst = jnp.reshape(nrm_ref[0, :], (RB, 1))
    nsb = jnp.reshape(nrm_ref[2, :], (RB, 1))
    nsr = jnp.reshape(nrm_ref[4, :], (RB, 1))
    fu = fu_ref[...]
    fi = fi_ref[...]
    dot = functools.partial(jnp.dot, preferred_element_type=jnp.float32,
                            precision=lax.Precision.HIGHEST)
    yt_ref[...] = dot(fu * nst, wt_ref[...])
    yb_ref[...] = dot(fi * nsb, wb_ref[...])
    yr_ref[...] = dot(fu * nsr, wr_ref[...])


def _a_call(nrm, fu, fi, wt, wb, wr):
    g = NP // RB
    return pl.pallas_call(
        _a_body,
        grid=(g,),
        in_specs=[
            pl.BlockSpec((6, RB), lambda i: (0, i)),
            pl.BlockSpec((RB, D), lambda i: (i, 0)),
            pl.BlockSpec((RB, D), lambda i: (i, 0)),
            pl.BlockSpec((D, D), lambda i: (0, 0)),
            pl.BlockSpec((D, D), lambda i: (0, 0)),
            pl.BlockSpec((D, D), lambda i: (0, 0)),
        ],
        out_specs=[pl.BlockSpec((RB, D), lambda i: (i, 0))] * 3,
        out_shape=[jax.ShapeDtypeStruct((NP, D), jnp.float32)] * 3,
    )(nrm, fu, fi, wt, wb, wr)


def _post(agg_ref, nrm_ref, b_ref):
    ndt = jnp.reshape(nrm_ref[1, :], (RB, 1))
    ndb = jnp.reshape(nrm_ref[3, :], (RB, 1))
    ndr = jnp.reshape(nrm_ref[5, :], (RB, 1))
    at = agg_ref[0, 0] + agg_ref[0, 1]
    ab = agg_ref[1, 0] + agg_ref[1, 1]
    ar = agg_ref[2, 0] + agg_ref[2, 1]
    ht = jnp.maximum(at * ndt + b_ref[0, :], 0.0)
    hb = jnp.maximum(ab * ndb + b_ref[1, :], 0.0)
    hr = jnp.maximum(ar * ndr + b_ref[2, :], 0.0)
    fu = (ht + hb) * 0.5
    return fu, hr


def _ba_body(agg_ref, nrm_ref, b_ref, wt_ref, wb_ref, wr_ref,
             fu_ref, fi_ref, yt_ref, yb_ref, yr_ref):
    fu, fi = _post(agg_ref, nrm_ref, b_ref)
    fu_ref[...] = fu
    fi_ref[...] = fi
    nst = jnp.reshape(nrm_ref[0, :], (RB, 1))
    nsb = jnp.reshape(nrm_ref[2, :], (RB, 1))
    nsr = jnp.reshape(nrm_ref[4, :], (RB, 1))
    dot = functools.partial(jnp.dot, preferred_element_type=jnp.float32,
                            precision=lax.Precision.HIGHEST)
    yt_ref[...] = dot(fu * nst, wt_ref[...])
    yb_ref[...] = dot(fi * nsb, wb_ref[...])
    yr_ref[...] = dot(fu * nsr, wr_ref[...])


def _ba_call(agg, nrm, b, wt, wb, wr):
    g = NP // RB
    return pl.pallas_call(
        _ba_body,
        grid=(g,),
        in_specs=[
            pl.BlockSpec((3, NC, RB, D), lambda i: (0, 0, i, 0)),
            pl.BlockSpec((6, RB), lambda i: (0, i)),
            pl.BlockSpec((3, D), lambda i: (0, 0)),
            pl.BlockSpec((D, D), lambda i: (0, 0)),
            pl.BlockSpec((D, D), lambda i: (0, 0)),
            pl.BlockSpec((D, D), lambda i: (0, 0)),
        ],
        out_specs=[pl.BlockSpec((RB, D), lambda i: (i, 0))] * 5,
        out_shape=([jax.ShapeDtypeStruct((NU, D), jnp.float32)] * 2
                   + [jax.ShapeDtypeStruct((NP, D), jnp.float32)] * 3),
    )(agg, nrm, b, wt, wb, wr)


def _b_body(agg_ref, nrm_ref, b_ref, fu_ref, fi_ref):
    fu, fi = _post(agg_ref, nrm_ref, b_ref)
    fu_ref[...] = fu
    fi_ref[...] = fi


def _b_call(agg, nrm, b):
    g = NP // RB
    return pl.pallas_call(
        _b_body,
        grid=(g,),
        in_specs=[
            pl.BlockSpec((3, NC, RB, D), lambda i: (0, 0, i, 0)),
            pl.BlockSpec((6, RB), lambda i: (0, i)),
            pl.BlockSpec((3, D), lambda i: (0, 0)),
        ],
        out_specs=[pl.BlockSpec((RB, D), lambda i: (i, 0))] * 2,
        out_shape=[jax.ShapeDtypeStruct((NU, D), jnp.float32)] * 2,
    )(agg, nrm, b)


def kernel(norm_edge_weight, user_embeddings, item_embeddings,
           Wt0, bt0, Wr0, br0, Wb0, bb0, Wt1, bt1, Wr1, br1, Wb1, bb1,
           trust_src, trust_dst, rate_src, rate_dst, rb_src, rb_dst):
    pad = NU + (jnp.arange(EPAD - E, dtype=jnp.int32) % (NP - NU))

    def idx(a):
        return jnp.concatenate([a.astype(jnp.int32), pad]).reshape(NCH2, CHUNK)
    ts, td = idx(trust_src), idx(trust_dst)
    bs, bd = idx(rb_src), idx(rb_dst)
    rs, rd = idx(rate_src), idx(rate_dst)
    cf = jnp.concatenate(
        [norm_edge_weight, jnp.zeros((EPAD - E,), jnp.float32)]
    ).reshape(NCH2, CHUNK)

    degp = _deg_call(ts, td, bs, bd, rs, rd).reshape(NW, 6, NP)
    nrm = _norms_call(degp)

    yt0, yb0, yr0 = _a_call(nrm, user_embeddings, item_embeddings,
                            Wt0, Wb0, Wr0)
    agg0 = _spmm_call(yt0, yb0, yr0, ts, td, bs, bd, rs, rd, cf)
    b0 = jnp.stack([bt0, bb0, br0])
    fu1, fi1, yt1, yb1, yr1 = _ba_call(agg0, nrm, b0, Wt1, Wb1, Wr1)

    agg1 = _spmm_call(yt1, yb1, yr1, ts, td, bs, bd, rs, rd, cf)
    b1 = jnp.stack([bt1, bb1, br1])
    fu2, fi2 = _b_call(agg1, nrm, b1)

    out_u = jnp.concatenate([user_embeddings, fu1, fu2], axis=1)
    out_i = jnp.concatenate([item_embeddings, fi1, fi2], axis=1)
    return (out_u, out_i)
